# baseline (speedup 1.0000x reference)
import functools

import jax
import jax.numpy as jnp
from jax import lax
from jax.experimental import pallas as pl
from jax.experimental.pallas import tpu as pltpu
from jax.experimental.pallas import tpu_sc as plsc

N, H = 2048, 1024
E, K, G, TG = 64, 8, 8, 4
EPG = E // G
F, FS = 256, 512
ROUTED_SCALE = 2.5
NEG = -1e30

TM = 128
NB = (N * K) // TM + E
P = NB * TM

NW = 32
TPW = N // NW
CH = 32


def _first_max_mask(a):
    m = jnp.max(a, axis=1, keepdims=True)
    ids = lax.broadcasted_iota(jnp.int32, a.shape, 1)
    cand = jnp.where(a == m, ids, a.shape[1])
    fid = jnp.min(cand, axis=1, keepdims=True)
    return ids == fid


def _cumsum_rows(a, n):
    s = 1
    while s < n:
        z = jnp.zeros((s,) + a.shape[1:], a.dtype)
        a = a + jnp.concatenate([z, a[:-s]], axis=0)
        s *= 2
    return a


def _cumsum_lanes(a, n):
    s = 1
    while s < n:
        z = jnp.zeros(a.shape[:1] + (s,) + a.shape[2:], a.dtype)
        a = a + jnp.concatenate([z, a[:, :-s]], axis=1)
        s *= 2
    return a


def _dot(a, b, dims):
    return lax.dot_general(a, b, (dims, ((), ())),
                           preferred_element_type=jnp.float32)


def _gate_body(x_ref, rw_ref, cb_ref, sg_ref, su_ref, sd_ref,
               ybase_ref, slots_ref, wts_ref, tokid_ref,
               be_ref, brows_ref, nb_ref):
    x = x_ref[...]
    logits = _dot(x, rw_ref[...], ((1,), (1,)))
    scores = jax.nn.sigmoid(logits)
    sfc = scores + cb_ref[...]

    gs_cols = []
    for g in range(G):
        sub = sfc[:, g * EPG:(g + 1) * EPG]
        first = _first_max_mask(sub)
        m1 = jnp.max(sub, axis=1, keepdims=True)
        m2 = jnp.max(jnp.where(first, NEG, sub), axis=1, keepdims=True)
        gs_cols.append(m1 + m2)
    gs = jnp.concatenate(gs_cols, axis=1)

    gmask = jnp.zeros((N, G), dtype=jnp.float32)
    for _ in range(TG):
        first = _first_max_mask(jnp.where(gmask > 0, NEG, gs))
        gmask = gmask + first.astype(jnp.float32)
    emask = jnp.concatenate(
        [jnp.broadcast_to(gmask[:, g:g + 1], (N, EPG)) for g in range(G)],
        axis=1)

    tmp = jnp.where(emask > 0, sfc, NEG)
    sel = jnp.zeros((N, E), dtype=jnp.float32)
    for _ in range(K):
        first = _first_max_mask(jnp.where(sel > 0, NEG, tmp))
        sel = sel + first.astype(jnp.float32)

    w = sel * scores
    denom = jnp.sum(w, axis=1, keepdims=True) + 1e-20
    comb = w / denom * ROUTED_SCALE

    counts = jnp.sum(sel, axis=0, keepdims=True)
    blocks = jnp.floor((counts + (TM - 1)) * (1.0 / TM))
    bstart = _cumsum_lanes(blocks, E) - blocks
    offset = bstart * TM
    rank = _cumsum_rows(sel, N) - sel
    dst = offset + rank

    rank_sel = _cumsum_lanes(sel, E) - sel
    slot_cols, wt_cols = [], []
    for k in range(K):
        mk = sel * (rank_sel == k).astype(jnp.float32)
        slot_cols.append(jnp.sum(dst * mk, axis=1, keepdims=True))
        wt_cols.append(jnp.sum(comb * mk, axis=1, keepdims=True))
    slots_ref[...] = jnp.concatenate(slot_cols, axis=1).astype(jnp.int32)
    wts_ref[...] = jnp.concatenate(wt_cols, axis=1)
    tokid_ref[...] = jnp.broadcast_to(
        lax.broadcasted_iota(jnp.int32, (N, K), 0), (N, K))

    iota_b = lax.broadcasted_iota(jnp.int32, (NB, E), 0).astype(jnp.float32)
    ind = jnp.logical_and(iota_b >= bstart, iota_b < bstart + blocks)
    indf = ind.astype(jnp.float32)
    e_ids = lax.broadcasted_iota(jnp.int32, (NB, E), 1).astype(jnp.float32)
    be_ref[...] = jnp.sum(indf * e_ids, axis=1, keepdims=True
                          ).astype(jnp.int32)
    rows_in_b = jnp.minimum(float(TM), counts - (iota_b - bstart) * TM)
    brows_ref[...] = jnp.sum(indf * rows_in_b, axis=1, keepdims=True
                             ).astype(jnp.int32)
    nb_ref[...] = jnp.sum(blocks, axis=1, keepdims=True).astype(jnp.int32)

    sg_ = _dot(x, sg_ref[...], ((1,), (1,)))
    su_ = _dot(x, su_ref[...], ((1,), (1,)))
    hs = su_ * (sg_ * jax.nn.sigmoid(sg_))
    ybase_ref[...] = x + _dot(hs, sd_ref[...], ((1,), (1,)))


@jax.jit
def _gate_tc(x, rw, cb, sg, su, sd):
    full = lambda shape: pl.BlockSpec(shape, lambda: (0,) * len(shape))
    return pl.pallas_call(
        _gate_body,
        grid=(),
        in_specs=[full((N, H)), full((E, H)), full((1, E)),
                  full((FS, H)), full((FS, H)), full((H, FS))],
        out_specs=[full((N, H)), full((N, K)), full((N, K)), full((N, K)),
                   full((NB, 1)), full((NB, 1)), full((1, 1))],
        out_shape=[
            jax.ShapeDtypeStruct((N, H), jnp.float32),
            jax.ShapeDtypeStruct((N, K), jnp.int32),
            jax.ShapeDtypeStruct((N, K), jnp.float32),
            jax.ShapeDtypeStruct((N, K), jnp.int32),
            jax.ShapeDtypeStruct((NB, 1), jnp.int32),
            jax.ShapeDtypeStruct((NB, 1), jnp.int32),
            jax.ShapeDtypeStruct((1, 1), jnp.int32),
        ],
    )(x, rw, cb, sg, su, sd)


def _sc_mesh():
    return plsc.VectorSubcoreMesh(core_axis_name="c", subcore_axis_name="s")


@jax.jit
def _sc_scatter(slots_f, tokid_f, wts_f):
    @functools.partial(
        pl.kernel, mesh=_sc_mesh(),
        out_type=[jax.ShapeDtypeStruct((P,), jnp.int32),
                  jax.ShapeDtypeStruct((P,), jnp.float32)],
        scratch_types=[pltpu.VMEM((4, 128), jnp.int32),
                       pltpu.VMEM((4, 128), jnp.int32),
                       pltpu.VMEM((4, 128), jnp.float32),
                       pltpu.SemaphoreType.DMA],
    )
    def k(slots_hbm, tokid_hbm, wts_hbm, tok_out, w_out,
          idx_v, tv_v, wv_v, sem):
        wid = lax.axis_index("s") * 2 + lax.axis_index("c")
        pltpu.sync_copy(slots_hbm.at[wid], idx_v)
        pltpu.sync_copy(tokid_hbm.at[wid], tv_v)
        pltpu.sync_copy(wts_hbm.at[wid], wv_v)
        for j in range(4):
            pltpu.async_copy(tv_v.at[j], tok_out.at[idx_v.at[j]], sem).wait()
            pltpu.async_copy(wv_v.at[j], w_out.at[idx_v.at[j]], sem).wait()

    return k(slots_f, tokid_f, wts_f)


@jax.jit
def _sc_gather(tok, x, nb1):
    nchunk = P // (NW * CH)

    @functools.partial(
        pl.kernel, mesh=_sc_mesh(),
        out_type=jax.ShapeDtypeStruct((P, H), jnp.float32),
        scratch_types=[pltpu.VMEM((CH,), jnp.int32),
                       pltpu.VMEM((CH,), jnp.int32),
                       pltpu.VMEM((CH, H), jnp.float32),
                       pltpu.VMEM((16,), jnp.int32),
                       pltpu.SemaphoreType.DMA],
    )
    def k(tok_hbm, x_hbm, nb_hbm, xd_hbm, idx_v, idx2_v, rows_v, nb_v, sem):
        wid = lax.axis_index("s") * 2 + lax.axis_index("c")
        pltpu.sync_copy(nb_hbm, nb_v)
        nrow = nb_v[...][0] * TM
        for ch in range(nchunk):
            base = (wid * nchunk + ch) * CH

            @pl.when(base < nrow)
            def _():
                pltpu.sync_copy(tok_hbm.at[pl.ds(base, CH)], idx_v)
                for l in range(CH // 16):
                    v = idx_v[pl.ds(l * 16, 16)]
                    idx2_v[pl.ds(l * 16, 16)] = lax.bitwise_and(v, N - 1)
                pltpu.async_copy(x_hbm.at[idx2_v], rows_v, sem).wait()
                pltpu.sync_copy(rows_v, xd_hbm.at[pl.ds(base, CH)])

    return k(tok, x, nb1)


def _ffn_body(be_ref, brows_ref, nb_ref, xd_ref, gw_ref, uw_ref, dw_ref,
              wr_ref, out_ref):
    b = pl.program_id(0)

    @pl.when(b < nb_ref[0])
    def _():
        x = xd_ref[...]
        valid = (lax.broadcasted_iota(jnp.int32, (TM, 1), 0)
                 < brows_ref[b])
        w = jnp.where(valid, wr_ref[0], 0.0)
        g = _dot(x, gw_ref[0], ((1,), (1,)))
        u = _dot(x, uw_ref[0], ((1,), (1,)))
        h = (g * jax.nn.sigmoid(g)) * u * w
        out_ref[...] = _dot(h, dw_ref[0], ((1,), (1,)))


@jax.jit
def _ffn_tc(be, brows, nb, xd, gw, uw, dw, wrow):
    def xd_map(b, be_r, br_r, nb_r):
        return (jnp.where(b < nb_r[0], b, 0), 0)

    def w_map(b, be_r, br_r, nb_r):
        return (jnp.where(b < nb_r[0], be_r[b], 0), 0, 0)

    def out_map(b, be_r, br_r, nb_r):
        return (jnp.where(b < nb_r[0], b, NB), 0)

    grid_spec = pltpu.PrefetchScalarGridSpec(
        num_scalar_prefetch=3,
        grid=(NB,),
        in_specs=[
            pl.BlockSpec((TM, H), xd_map),
            pl.BlockSpec((1, F, H), w_map),
            pl.BlockSpec((1, F, H), w_map),
            pl.BlockSpec((1, H, F), w_map),
            pl.BlockSpec((1, TM, 1), lambda b, *_: (b, 0, 0)),
        ],
        out_specs=pl.BlockSpec((TM, H), out_map),
    )
    return pl.pallas_call(
        _ffn_body,
        grid_spec=grid_spec,
        out_shape=jax.ShapeDtypeStruct((P + TM, H), jnp.float32),
        compiler_params=pltpu.CompilerParams(
            dimension_semantics=("arbitrary",),
        ),
    )(be, brows, nb, xd, gw, uw, dw, wrow)


@jax.jit
def _sc_combine(ybase, ybuf, slots8):
    @functools.partial(
        pl.kernel, mesh=_sc_mesh(),
        out_type=jax.ShapeDtypeStruct((N, H), jnp.float32),
        scratch_types=[pltpu.VMEM((8, 64), jnp.int32),
                       pltpu.VMEM((64, H), jnp.float32),
                       pltpu.VMEM((8, H), jnp.float32),
                       pltpu.SemaphoreType.DMA],
    )
    def k(ybase_hbm, ybuf_hbm, slots_hbm, y_hbm, sl_v, rows_v, acc_v, sem):
        wid = lax.axis_index("s") * 2 + lax.axis_index("c")
        base_t = wid * TPW
        pltpu.sync_copy(slots_hbm.at[wid], sl_v)
        for t8 in range(TPW // 8):
            cp = pltpu.async_copy(ybuf_hbm.at[sl_v.at[t8]], rows_v, sem)
            pltpu.sync_copy(ybase_hbm.at[pl.ds(base_t + t8 * 8, 8)], acc_v)
            cp.wait()
            for t in range(8):
                def cbody(c, _):
                    off = c * 16
                    s = acc_v[t, pl.ds(off, 16)]
                    for kk in range(K):
                        s = s + rows_v[t * 8 + kk, pl.ds(off, 16)]
                    acc_v[t, pl.ds(off, 16)] = s
                    return 0
                lax.fori_loop(0, H // 16, cbody, 0)
            pltpu.sync_copy(acc_v, y_hbm.at[pl.ds(base_t + t8 * 8, 8)])

    return k(ybase, ybuf, slots8)


def kernel(hidden_states, router_w, corr_bias, gate_w, up_w, down_w,
           s_gate, s_up, s_down):
    Bq, Sq, Hq = hidden_states.shape
    x = hidden_states.reshape(N, H)
    ybase, slots, wts, tokid, be, brows, nb = _gate_tc(
        x, router_w, corr_bias.reshape(1, E), s_gate, s_up, s_down)
    slots_f = slots.reshape(NW, 4, 128)
    tokid_f = tokid.reshape(NW, 4, 128)
    wts_f = wts.reshape(NW, 4, 128)
    tok, wrow = _sc_scatter(slots_f, tokid_f, wts_f)
    nb1 = jnp.broadcast_to(nb.reshape(1), (16,))
    xd = _sc_gather(tok, x, nb1)
    ybuf = _ffn_tc(be.reshape(NB), brows.reshape(NB), nb.reshape(1),
                   xd, gate_w, up_w, down_w, wrow.reshape(NB, TM, 1))
    y = _sc_combine(ybase, ybuf, slots.reshape(NW, 8, 64))
    return y.reshape(Bq, Sq, Hq)
